# baseline (speedup 1.0000x reference)
import functools

import jax
import jax.numpy as jnp
from jax import lax
from jax.experimental import pallas as pl
from jax.experimental.pallas import tpu as pltpu
from jax.experimental.pallas import tpu_sc as plsc

BATCH = 16384
EMBED_DIM = 64
LANES = 16
CHUNK = 128


def _make_kernel():
    info = plsc.get_sparse_core_info()
    nc, ns = info.num_cores, info.num_subcores
    nw = nc * ns
    per_w = BATCH // nw
    n_chunks = per_w // CHUNK

    mesh = plsc.VectorSubcoreMesh(core_axis_name="c", subcore_axis_name="s")

    @functools.partial(
        pl.kernel,
        mesh=mesh,
        compiler_params=pltpu.CompilerParams(
            needs_layout_passes=False, use_tc_tiling_on_sc=False),
        out_type=jax.ShapeDtypeStruct((BATCH,), jnp.float32),
        scratch_types=[
            pltpu.VMEM((n_chunks, CHUNK), jnp.int32),
            pltpu.VMEM((n_chunks, CHUNK), jnp.int32),
            pltpu.VMEM((n_chunks, CHUNK), jnp.int32),
            pltpu.VMEM((per_w, EMBED_DIM), jnp.float32),
            pltpu.VMEM((per_w, EMBED_DIM), jnp.float32),
            pltpu.VMEM((per_w, EMBED_DIM), jnp.float32),
            pltpu.VMEM((per_w,), jnp.float32),
            pltpu.SemaphoreType.DMA,
        ],
    )
    def distmult(head_hbm, rel_hbm, tail_hbm, ent_hbm, relemb_hbm, out_hbm,
                 idx_h, idx_r, idx_t, rows_h, rows_r, rows_t, out_v, sem):
        wid = lax.axis_index("s") * nc + lax.axis_index("c")
        base = wid * per_w

        for j in range(n_chunks):
            off = base + j * CHUNK
            pltpu.sync_copy(head_hbm.at[pl.ds(off, CHUNK)], idx_h.at[j])
            pltpu.sync_copy(rel_hbm.at[pl.ds(off, CHUNK)], idx_r.at[j])
            pltpu.sync_copy(tail_hbm.at[pl.ds(off, CHUNK)], idx_t.at[j])

        copies = []
        for j in range(n_chunks):
            rsl = pl.ds(j * CHUNK, CHUNK)
            copies.append(pltpu.async_copy(
                ent_hbm.at[idx_h.at[j]], rows_h.at[rsl], sem))
            copies.append(pltpu.async_copy(
                relemb_hbm.at[idx_r.at[j]], rows_r.at[rsl], sem))
            copies.append(pltpu.async_copy(
                ent_hbm.at[idx_t.at[j]], rows_t.at[rsl], sem))
        for c in copies:
            c.wait()

        iota16 = lax.iota(jnp.int32, 16)

        def group(g, _):
            gacc = jnp.zeros((LANES,), jnp.float32)
            for k in range(LANES):
                i = g * LANES + k
                acc = None
                for c in range(EMBED_DIM // LANES):
                    dsl = pl.ds(c * LANES, LANES)
                    p = rows_h[i, dsl] * rows_r[i, dsl] * rows_t[i, dsl]
                    acc = p if acc is None else acc + p
                s = jnp.sum(acc)
                gacc = gacc + jnp.where(iota16 == k, s, 0.0)
            out_v[pl.ds(g * LANES, LANES)] = gacc
            return 0

        lax.fori_loop(0, per_w // LANES, group, 0)

        pltpu.sync_copy(out_v, out_hbm.at[pl.ds(base, per_w)])

    return distmult


_distmult = _make_kernel()


def kernel(head, relation, tail, entity_embeddings, relation_embeddings):
    return _distmult(head, relation, tail, entity_embeddings,
                     relation_embeddings)
